# baseline (speedup 1.0000x reference)
import functools

import jax
import jax.numpy as jnp
from jax import lax
from jax.experimental import pallas as pl
from jax.experimental.pallas import tpu as pltpu
from jax.experimental.pallas import tpu_sc as plsc

NC = 2
NS = 16
L = 16
NW = NC * NS
B = 16384
D = 64
W = 2 * D
BPW = B // NW
PASS = 256
NPASS = BPW // PASS
GROUPS = PASS // L


@functools.partial(
    pl.kernel,
    out_type=jax.ShapeDtypeStruct((B,), jnp.float32),
    mesh=plsc.VectorSubcoreMesh(core_axis_name="c", subcore_axis_name="s"),
    compiler_params=pltpu.CompilerParams(needs_layout_passes=False,
                                         use_tc_tiling_on_sc=False),
    scratch_types=[
        pltpu.VMEM((2 * BPW,), jnp.int32),
        pltpu.VMEM((BPW,), jnp.int32),
        pltpu.VMEM((BPW,), jnp.int32),
        pltpu.VMEM((PASS, W), jnp.float32),
        pltpu.VMEM((PASS, W), jnp.float32),
        pltpu.VMEM((BPW,), jnp.float32),
        pltpu.VMEM((BPW,), jnp.float32),
        pltpu.VMEM((1,), jnp.float32),
        pltpu.VMEM((BPW,), jnp.float32),
        pltpu.VMEM((L, L + 1), jnp.float32),
        pltpu.SemaphoreType.DMA,
        pltpu.SemaphoreType.DMA,
    ],
)
def _mf_kernel(x_hbm, p_hbm, q_hbm, bu_hbm, bi_hbm, alfa_hbm, out_hbm,
               x_v, u_par, i_par, pu_v, qi_v, bu_v, bi_v, alfa_v,
               out_v, tr_v, sem, sem_b):
    wid = lax.axis_index("s") * NC + lax.axis_index("c")
    base = wid * BPW
    iota = lax.iota(jnp.int32, L)

    pltpu.sync_copy(x_hbm.at[pl.ds(base * 2, 2 * BPW)], x_v)
    pltpu.sync_copy(alfa_hbm, alfa_v)
    alfa_s = plsc.load_gather(alfa_v, [iota * 0])

    for p in range(NPASS):
        copies, bcopies = [], []
        for g in range(GROUPS):
            e0 = p * PASS + g * L
            off = 2 * e0
            u16 = plsc.load_gather(x_v, [off + 2 * iota])
            i16 = plsc.load_gather(x_v, [off + 2 * iota + 1])
            u_par[pl.ds(e0, L)] = (u16 & 1) * D
            i_par[pl.ds(e0, L)] = (i16 & 1) * D
            sl = pl.ds(g * L, L)
            esl = pl.ds(e0, L)
            copies.append(pltpu.async_copy(p_hbm.at[u16 >> 1], pu_v.at[sl], sem))
            copies.append(pltpu.async_copy(q_hbm.at[i16 >> 1], qi_v.at[sl], sem))
            bcopies.append(pltpu.async_copy(bu_hbm.at[u16], bu_v.at[esl], sem_b))
            bcopies.append(pltpu.async_copy(bi_hbm.at[i16], bi_v.at[esl], sem_b))
        for c in copies:
            c.wait()
        for c in bcopies:
            c.wait()

        def group(g, carry, p=p):
            b0 = g * L
            eb = p * PASS + b0
            up16 = u_par[pl.ds(eb, L)]
            ip16 = i_par[pl.ds(eb, L)]
            for r in range(L):
                row = b0 + r
                uo = up16[r]
                io = ip16[r]
                acc = pu_v[row, pl.ds(uo, L)] * qi_v[row, pl.ds(io, L)]
                for c in range(1, D // L):
                    acc = acc + (pu_v[row, pl.ds(uo + c * L, L)]
                                 * qi_v[row, pl.ds(io + c * L, L)])
                tr_v[r, pl.ds(0, L)] = acc
            tot = plsc.load_gather(tr_v, [iota, iota * 0])
            for l in range(1, L):
                tot = tot + plsc.load_gather(tr_v, [iota, jnp.full((L,), l, jnp.int32)])
            out_v[pl.ds(eb, L)] = tot + bu_v[pl.ds(eb, L)] + bi_v[pl.ds(eb, L)] + alfa_s
            return carry

        lax.fori_loop(0, GROUPS, group, 0)

    pltpu.sync_copy(out_v, out_hbm.at[pl.ds(base, BPW)])


def kernel(x, P, Q, beta_u, beta_i, alfa):
    xf = x.reshape(-1).astype(jnp.int32)
    nrow = P.shape[0] // 2
    return _mf_kernel(xf, P.reshape(nrow, W), Q.reshape(nrow, W),
                      beta_u.reshape(-1), beta_i.reshape(-1),
                      alfa.reshape(-1))
